# Initial kernel scaffold; baseline (speedup 1.0000x reference)
import jax
import jax.numpy as jnp
from jax.experimental import pallas as pl


def kernel(node_attr, edge_index, edge_feature, edge_vector, W1, b1, W2, b2, W, b, Wo, bo, Wo1, bo1):
    raise NotImplementedError("write your pallas kernel here")


# speedup vs baseline: 16.8325x; 16.8325x over previous
import functools
import numpy as np
import jax
import jax.numpy as jnp
from jax import lax
from jax.experimental import pallas as pl
from jax.experimental.pallas import tpu as pltpu
from jax.experimental.pallas import tpu_sc as plsc

HEADS = 4
HDIM = 16
F = HEADS * HDIM
PC = 128
CH = 128
NC = 2
NS = 16
NW = NC * NS
NPASS = 5
H4 = 10112
NPAD = NPASS * H4
SROWS = H4 + NS * 8

_S_NP = np.kron(np.eye(HEADS, dtype=np.float32), np.ones((HDIM, 1), np.float32))


def _elu(x):
    return jnp.where(x > 0, x, jnp.exp(jnp.minimum(x, 0.0)) - 1.0)


def _nodeG_body(ev_ref, W1_ref, b1_ref, W2_ref, b2_ref, W_ref, b_ref, out_ref):
    ev = ev_ref[...]
    h1 = _elu(ev * W1_ref[...] + b1_ref[...])
    h2 = jnp.dot(h1, W2_ref[...], preferred_element_type=jnp.float32) + b2_ref[...]
    g = jnp.dot(h2, W_ref[...], preferred_element_type=jnp.float32) + b_ref[...]
    out_ref[...] = jnp.concatenate([g, jnp.zeros_like(g)], axis=1)


def _payload_body(sg_ref, dg_ref, ef_ref, W2_ref, b2_ref, W_ref, b_ref,
                  S_ref, ST_ref, out_ref):
    sg = sg_ref[...][:, :F]
    dg = dg_ref[...][:, :F]
    ea = _elu(jnp.dot(ef_ref[...], W2_ref[...], preferred_element_type=jnp.float32)
              + b2_ref[...])
    E = jnp.dot(ea, W_ref[...], preferred_element_type=jnp.float32) + b_ref[...]
    pre = jnp.dot(sg * dg * E, S_ref[...], preferred_element_type=jnp.float32) * 0.25
    sc = jnp.exp(jnp.minimum(jnp.maximum(pre, -5.0), 5.0))
    msg = sg * jnp.dot(sc, ST_ref[...], preferred_element_type=jnp.float32)
    pad = jnp.zeros((msg.shape[0], PC - F - HEADS), jnp.float32)
    out_ref[...] = jnp.concatenate([msg, sc, pad], axis=1)


def _final_body(a_ref, b2p_ref, ST_ref, Wo_ref, bo_ref, Wo1_ref, bo1_ref, out_ref):
    wvz = a_ref[...] + b2p_ref[...]
    wv = wvz[:, :F]
    z = wvz[:, F:F + HEADS]
    zb = jnp.dot(z, ST_ref[...], preferred_element_type=jnp.float32)
    ho = wv / (zb + 1e-6)
    hm = _elu(jnp.dot(ho, Wo_ref[...], preferred_element_type=jnp.float32) + bo_ref[...])
    out_ref[...] = jnp.dot(hm, Wo1_ref[...], preferred_element_type=jnp.float32) + bo1_ref[...]


def kernel(node_attr, edge_index, edge_feature, edge_vector,
           W1, b1, W2, b2, W, b, Wo, bo, Wo1, bo1):
    Nn = edge_vector.shape[0]
    M = edge_index.shape[1]

    K = -(-M // (NW * CH))
    Mpad = NW * K * CH

    evp = jnp.pad(edge_vector, (0, NPAD - Nn)).reshape(NPAD, 1)
    b1r, b2r, br = b1.reshape(1, -1), b2.reshape(1, -1), b.reshape(1, -1)
    bor, bo1r = bo.reshape(1, -1), bo1.reshape(1, -1)
    Smat = jnp.asarray(_S_NP)
    STmat = jnp.asarray(_S_NP.T)

    src = edge_index[0]
    dst = edge_index[1]
    src_g = jnp.pad(src, (0, Mpad - M))
    dst_g = jnp.pad(dst, (0, Mpad - M))
    dst_s = jnp.pad(dst, (0, Mpad - M), constant_values=NPAD)
    idx_p = [jnp.where((dst_s >= p * H4) & (dst_s < (p + 1) * H4),
                       dst_s - p * H4, H4).astype(jnp.int32)
             for p in range(NPASS)]
    idx4 = jnp.stack(idx_p).reshape(NPASS, NW, K, CH)
    sidx3 = src_g.reshape(NW, K, CH)
    didx3 = dst_g.reshape(NW, K, CH)
    efp = jnp.pad(edge_feature, ((0, Mpad - M), (0, 0)))
    zrows = jnp.zeros((SROWS, PC), jnp.float32)

    BN = 640
    G = pl.pallas_call(
        _nodeG_body,
        grid=(NPAD // BN,),
        in_specs=[
            pl.BlockSpec((BN, 1), lambda i: (i, 0)),
            pl.BlockSpec((1, 4), lambda i: (0, 0)),
            pl.BlockSpec((1, 4), lambda i: (0, 0)),
            pl.BlockSpec((4, 16), lambda i: (0, 0)),
            pl.BlockSpec((1, 16), lambda i: (0, 0)),
            pl.BlockSpec((16, F), lambda i: (0, 0)),
            pl.BlockSpec((1, F), lambda i: (0, 0)),
        ],
        out_specs=pl.BlockSpec((BN, PC), lambda i: (i, 0)),
        out_shape=jax.ShapeDtypeStruct((NPAD, PC), jnp.float32),
    )(evp, W1, b1r, W2, b2r, W, br)

    mesh = plsc.VectorSubcoreMesh(core_axis_name="c", subcore_axis_name="s",
                                  num_cores=NC)

    @functools.partial(
        pl.kernel, mesh=mesh,
        out_type=(jax.ShapeDtypeStruct((Mpad, PC), jnp.float32),
                  jax.ShapeDtypeStruct((Mpad, PC), jnp.float32)),
        scratch_types=[
            pltpu.VMEM((K, CH), jnp.int32),
            pltpu.VMEM((K, CH), jnp.int32),
            pltpu.VMEM((CH, PC), jnp.float32),
            pltpu.VMEM((CH, PC), jnp.float32),
            pltpu.SemaphoreType.DMA,
            pltpu.SemaphoreType.DMA,
        ],
    )
    def _gather(G_hbm, si_hbm, di_hbm, os_hbm, od_hbm,
                si_v, di_v, sr_v, dr_v, sem1, sem2):
        wid = lax.axis_index("s") * NC + lax.axis_index("c")
        pltpu.sync_copy(si_hbm.at[wid], si_v)
        pltpu.sync_copy(di_hbm.at[wid], di_v)

        def body(j, carry):
            cs = pltpu.async_copy(G_hbm.at[si_v.at[j]], sr_v, sem1)
            cd = pltpu.async_copy(G_hbm.at[di_v.at[j]], dr_v, sem2)
            cs.wait()
            cd.wait()
            base = (wid * K + j) * CH
            pltpu.sync_copy(sr_v, os_hbm.at[pl.ds(base, CH)])
            pltpu.sync_copy(dr_v, od_hbm.at[pl.ds(base, CH)])
            return carry

        lax.fori_loop(0, K, body, 0)

    srcG, dstG = _gather(G, sidx3, didx3)

    BE = 512
    pay = pl.pallas_call(
        _payload_body,
        grid=(Mpad // BE,),
        in_specs=[
            pl.BlockSpec((BE, PC), lambda i: (i, 0)),
            pl.BlockSpec((BE, PC), lambda i: (i, 0)),
            pl.BlockSpec((BE, 4), lambda i: (i, 0)),
            pl.BlockSpec((4, 16), lambda i: (0, 0)),
            pl.BlockSpec((1, 16), lambda i: (0, 0)),
            pl.BlockSpec((16, F), lambda i: (0, 0)),
            pl.BlockSpec((1, F), lambda i: (0, 0)),
            pl.BlockSpec((F, 4), lambda i: (0, 0)),
            pl.BlockSpec((4, F), lambda i: (0, 0)),
        ],
        out_specs=pl.BlockSpec((BE, PC), lambda i: (i, 0)),
        out_shape=jax.ShapeDtypeStruct((Mpad, PC), jnp.float32),
    )(srcG, dstG, efp, W2, b2r, W, br, Smat, STmat)
    pay3 = pay.reshape(NW * K, CH, PC)

    @functools.partial(
        pl.kernel, mesh=mesh,
        out_type=jax.ShapeDtypeStruct((NC, NPAD, PC), jnp.float32),
        scratch_types=[
            pltpu.VMEM((K, CH), jnp.int32),
            pltpu.VMEM((CH, PC), jnp.float32),
            pltpu.VMEM_SHARED((SROWS, PC), jnp.float32),
        ],
    )
    def _scatter(pay_hbm, i4_hbm, z_hbm, out_hbm, idx_v, pay_v, shared):
        cid = lax.axis_index("c")
        sid = lax.axis_index("s")
        wid = sid * NC + cid
        ZR = SROWS // NS
        CR = H4 // NS
        for p in range(NPASS):
            pltpu.sync_copy(z_hbm.at[pl.ds(sid * ZR, ZR)],
                            shared.at[pl.ds(sid * ZR, ZR)])
            plsc.subcore_barrier()
            pltpu.sync_copy(i4_hbm.at[p, wid], idx_v)

            def body(j, carry):
                pltpu.sync_copy(pay_hbm.at[wid * K + j], pay_v)
                pltpu.sync_copy(pay_v, shared.at[idx_v.at[j]], add=True)
                return carry

            lax.fori_loop(0, K, body, 0)
            plsc.subcore_barrier()
            pltpu.sync_copy(shared.at[pl.ds(sid * CR, CR)],
                            out_hbm.at[cid, pl.ds(p * H4 + sid * CR, CR)])
            plsc.subcore_barrier()

    wvz = _scatter(pay3, idx4, zrows)

    res = pl.pallas_call(
        _final_body,
        grid=(NPAD // BN,),
        in_specs=[
            pl.BlockSpec((BN, PC), lambda i: (i, 0)),
            pl.BlockSpec((BN, PC), lambda i: (i, 0)),
            pl.BlockSpec((4, F), lambda i: (0, 0)),
            pl.BlockSpec((F, 16), lambda i: (0, 0)),
            pl.BlockSpec((1, 16), lambda i: (0, 0)),
            pl.BlockSpec((16, 1), lambda i: (0, 0)),
            pl.BlockSpec((1, 1), lambda i: (0, 0)),
        ],
        out_specs=pl.BlockSpec((BN, 1), lambda i: (i, 0)),
        out_shape=jax.ShapeDtypeStruct((NPAD, 1), jnp.float32),
    )(wvz[0], wvz[1], STmat, Wo, bor, Wo1, bo1r)

    return res[:Nn].reshape(-1, 3)
